# baseline (speedup 1.0000x reference)
import functools

import jax
import jax.numpy as jnp
from jax import lax
from jax.experimental import pallas as pl
from jax.experimental.pallas import tpu as pltpu
from jax.experimental.pallas import tpu_sc as plsc

N_HOST = 10000
N_FLOW = 50000
E = 320000
D = 128
H = 256
OUT = 64

NC = 2
NS = 16

_MESH = plsc.VectorSubcoreMesh(
    core_axis_name="c", subcore_axis_name="s", num_cores=NC, num_subcores=NS)


def _make_counts(C, P):
    ept = E // NS
    nfull = ept // 128
    tail = ept - nfull * 128
    z_rpt = (C + 128) // NS
    d_rpt = C // NS

    def body(dst_hbm, out_hbm, dst_st, dfire, dfire_t, ones, zsrc, acc, sem):
        c = lax.axis_index("c")
        s = lax.axis_index("s")
        pltpu.sync_copy(dst_hbm.at[pl.ds(s * ept, ept)], dst_st)
        one16 = jnp.ones((16,), jnp.float32)
        zero16 = jnp.zeros((16,), jnp.float32)
        for r in range(128):
            ones[r, pl.ds(0, 16)] = one16
        for r in range(8):
            zsrc[r, pl.ds(0, 16)] = zero16
        it16 = lax.iota(jnp.int32, 16)

        for p in range(P):
            lo = (c * P + p) * C
            hi = lo + C
            _counts_pass(acc, dst_st, dfire, dfire_t, ones, zsrc, out_hbm,
                         s, lo, hi)

    def _counts_pass(acc, dst_st, dfire, dfire_t, ones, zsrc, out_hbm, s, lo, hi):
        it16 = lax.iota(jnp.int32, 16)
        one16 = jnp.ones((16,), jnp.float32)
        for r in range(128):
            ones[r, pl.ds(0, 16)] = one16

        def zb(i, _):
            pltpu.sync_copy(zsrc, acc.at[pl.ds(s * z_rpt + i * 8, 8)])
            return 0
        lax.fori_loop(0, z_rpt // 8, zb, 0)
        plsc.subcore_barrier()

        def build(df, ob, ngrp):
            for k in range(ngrp):
                d16 = dst_st[pl.ds(ob + k * 16, 16)]
                inr = (d16 >= lo) & (d16 < hi)
                df[pl.ds(k * 16, 16)] = jnp.where(inr, d16 - lo, C + it16)

        def blk(i, _):
            build(dfire, i * 128, 8)
            pltpu.sync_copy(ones, acc.at[dfire], add=True)
            return 0
        lax.fori_loop(0, nfull, blk, 0)
        if tail:
            build(dfire_t, nfull * 128, tail // 16)
            pltpu.sync_copy(ones.at[pl.ds(0, tail)], acc.at[dfire_t], add=True)
        plsc.subcore_barrier()

        dnfull = d_rpt // 128
        dtail = d_rpt - dnfull * 128
        for i in range(dnfull):
            r = s * d_rpt + i * 128
            pltpu.sync_copy(acc.at[pl.ds(r, 128)], ones)
            pltpu.sync_copy(ones, out_hbm.at[pl.ds(lo + r, 128)])
        if dtail:
            r = s * d_rpt + dnfull * 128
            pltpu.sync_copy(acc.at[pl.ds(r, dtail)], ones.at[pl.ds(0, dtail)])
            pltpu.sync_copy(ones.at[pl.ds(0, dtail)],
                            out_hbm.at[pl.ds(lo + r, dtail)])
        plsc.subcore_barrier()

    if tail % 16 or C % 128 or z_rpt % 8 or d_rpt % 8:
        raise ValueError("bad counts geometry")

    return pl.kernel(
        body,
        out_type=jax.ShapeDtypeStruct((NC * P * C, 16), jnp.float32),
        mesh=_MESH,
        scratch_types=[
            pltpu.VMEM((ept,), jnp.int32),
            pltpu.VMEM((128,), jnp.int32),
            pltpu.VMEM((max(tail, 16),), jnp.int32),
            pltpu.VMEM((128, 16), jnp.float32),
            pltpu.VMEM((8, 16), jnp.float32),
            pltpu.VMEM_SHARED((C + 128, 16), jnp.float32),
            pltpu.SemaphoreType.DMA,
        ],
    )


W = 128
FIRE = 128


def _make_seg_sum(C, P):
    ept = E // NS
    npair = ept // (2 * FIRE)
    tail = ept - npair * 2 * FIRE
    rows_out = NC * P * C
    if C % 128 or tail % 16 or tail >= FIRE:
        raise ValueError("bad segsum geometry")
    z_rpt = (C + 128) // NS
    d_rpt = C // NS

    def body(x_hbm, src_hbm, dst_hbm, out_hbm,
             src_st, dst_st, dfA, dfB, dfT, rowsA, rowsB, zsrc, acc,
             semA, semB):
        c = lax.axis_index("c")
        s = lax.axis_index("s")
        pltpu.sync_copy(src_hbm.at[pl.ds(s * ept, ept)], src_st)
        pltpu.sync_copy(dst_hbm.at[pl.ds(s * ept, ept)], dst_st)
        zero16 = jnp.zeros((16,), jnp.float32)
        for r in range(8):
            for k in range(W // 16):
                zsrc[r, pl.ds(k * 16, 16)] = zero16
        it16 = lax.iota(jnp.int32, 16)

        for p in range(P):
            lo = (c * P + p) * C
            hi = lo + C

            def zb(i, _):
                pltpu.sync_copy(zsrc, acc.at[pl.ds(s * z_rpt + i * 8, 8)])
                return 0
            lax.fori_loop(0, z_rpt // 8, zb, 0)
            plsc.subcore_barrier()

            def build(df, ob, ngrp):
                for k in range(ngrp):
                    d16 = dst_st[pl.ds(ob + k * 16, 16)]
                    inr = (d16 >= lo) & (d16 < hi)
                    df[pl.ds(k * 16, 16)] = jnp.where(inr, d16 - lo, C + it16)

            def pair(j, _):
                ob0 = j * (2 * FIRE)
                ob1 = ob0 + FIRE
                g0 = pltpu.async_copy(
                    x_hbm.at[src_st.at[pl.ds(ob0, FIRE)]], rowsA, semA)
                g1 = pltpu.async_copy(
                    x_hbm.at[src_st.at[pl.ds(ob1, FIRE)]], rowsB, semB)
                build(dfA, ob0, FIRE // 16)
                g0.wait()
                pltpu.sync_copy(rowsA, acc.at[dfA], add=True)
                build(dfB, ob1, FIRE // 16)
                g1.wait()
                pltpu.sync_copy(rowsB, acc.at[dfB], add=True)
                return 0
            lax.fori_loop(0, npair, pair, 0)
            if tail:
                ob = npair * 2 * FIRE
                gt = pltpu.async_copy(
                    x_hbm.at[src_st.at[pl.ds(ob, tail)]],
                    rowsA.at[pl.ds(0, tail)], semA)
                build(dfT, ob, tail // 16)
                gt.wait()
                pltpu.sync_copy(rowsA.at[pl.ds(0, tail)], acc.at[dfT], add=True)
            plsc.subcore_barrier()

            dnfull = d_rpt // FIRE
            dtail = d_rpt - dnfull * FIRE
            for i in range(dnfull):
                r = s * d_rpt + i * FIRE
                pltpu.sync_copy(acc.at[pl.ds(r, FIRE)], rowsA)
                pltpu.sync_copy(rowsA, out_hbm.at[pl.ds(lo + r, FIRE)])
            if dtail:
                r = s * d_rpt + dnfull * FIRE
                pltpu.sync_copy(acc.at[pl.ds(r, dtail)], rowsA.at[pl.ds(0, dtail)])
                pltpu.sync_copy(rowsA.at[pl.ds(0, dtail)],
                                out_hbm.at[pl.ds(lo + r, dtail)])
            plsc.subcore_barrier()

    return pl.kernel(
        body,
        out_type=jax.ShapeDtypeStruct((rows_out, W), jnp.float32),
        mesh=_MESH,
        scratch_types=[
            pltpu.VMEM((ept,), jnp.int32),
            pltpu.VMEM((ept,), jnp.int32),
            pltpu.VMEM((FIRE,), jnp.int32),
            pltpu.VMEM((FIRE,), jnp.int32),
            pltpu.VMEM((max(tail, 16),), jnp.int32),
            pltpu.VMEM((FIRE, W), jnp.float32),
            pltpu.VMEM((FIRE, W), jnp.float32),
            pltpu.VMEM((8, W), jnp.float32),
            pltpu.VMEM_SHARED((C + 128, W), jnp.float32),
            pltpu.SemaphoreType.DMA,
            pltpu.SemaphoreType.DMA,
        ],
    )


def _combine1_body(relu, s_ref, c_ref, x_ref, wl_ref, wr_ref, b_ref, o_ref):
    scale = 1.0 / jnp.maximum(c_ref[:, 0:1], 1.0)
    a = s_ref[...] * scale
    y = (jnp.dot(a, wl_ref[...], preferred_element_type=jnp.float32)
         + jnp.dot(x_ref[...], wr_ref[...], preferred_element_type=jnp.float32)
         + b_ref[...])
    o_ref[...] = jnp.where(y > 0, y, 0.01 * y) if relu else y


def _combine1(S_pad, cnt_pad, X, Wl, Wr, b, relu, BM=400):
    M, K = X.shape
    return pl.pallas_call(
        functools.partial(_combine1_body, relu),
        grid=(M // BM,),
        in_specs=[
            pl.BlockSpec((BM, K), lambda i: (i, 0)),
            pl.BlockSpec((BM, 16), lambda i: (i, 0)),
            pl.BlockSpec((BM, K), lambda i: (i, 0)),
            pl.BlockSpec((K, H), lambda i: (0, 0)),
            pl.BlockSpec((K, H), lambda i: (0, 0)),
            pl.BlockSpec((H,), lambda i: (0,)),
        ],
        out_specs=pl.BlockSpec((BM, H), lambda i: (i, 0)),
        out_shape=jax.ShapeDtypeStruct((M, H), jnp.float32),
    )(S_pad, cnt_pad, X, Wl, Wr, b)


def _combine2_body(relu, sa_ref, sb_ref, c_ref, x_ref, wl_ref, wr_ref, b_ref,
                   o_ref):
    scale = 1.0 / jnp.maximum(c_ref[:, 0:1], 1.0)
    a = jnp.concatenate([sa_ref[...], sb_ref[...]], axis=1) * scale
    y = (jnp.dot(a, wl_ref[...], preferred_element_type=jnp.float32)
         + jnp.dot(x_ref[...], wr_ref[...], preferred_element_type=jnp.float32)
         + b_ref[...])
    o_ref[...] = jnp.where(y > 0, y, 0.01 * y) if relu else y


def _combine2(Sa, Sb, cnt_pad, X, Wl, Wr, b, relu, BM=400):
    M, K = X.shape
    return pl.pallas_call(
        functools.partial(_combine2_body, relu),
        grid=(M // BM,),
        in_specs=[
            pl.BlockSpec((BM, W), lambda i: (i, 0)),
            pl.BlockSpec((BM, W), lambda i: (i, 0)),
            pl.BlockSpec((BM, 16), lambda i: (i, 0)),
            pl.BlockSpec((BM, K), lambda i: (i, 0)),
            pl.BlockSpec((K, H), lambda i: (0, 0)),
            pl.BlockSpec((K, H), lambda i: (0, 0)),
            pl.BlockSpec((H,), lambda i: (0,)),
        ],
        out_specs=pl.BlockSpec((BM, H), lambda i: (i, 0)),
        out_shape=jax.ShapeDtypeStruct((M, H), jnp.float32),
    )(Sa, Sb, cnt_pad, X, Wl, Wr, b)


def _final_body(x_ref, w_ref, b_ref, o_ref):
    o_ref[...] = jnp.dot(x_ref[...], w_ref[...],
                         preferred_element_type=jnp.float32) + b_ref[...]


def _final(x, w, b, BM=1000):
    M, K = x.shape
    return pl.pallas_call(
        _final_body,
        grid=(M // BM,),
        in_specs=[
            pl.BlockSpec((BM, K), lambda i: (i, 0)),
            pl.BlockSpec((K, OUT), lambda i: (0, 0)),
            pl.BlockSpec((OUT,), lambda i: (0,)),
        ],
        out_specs=pl.BlockSpec((BM, OUT), lambda i: (i, 0)),
        out_shape=jax.ShapeDtypeStruct((M, OUT), jnp.float32),
    )(x, w, b)


_counts_part = _make_counts(C=5120, P=1)
_counts_host = _make_counts(C=5120, P=1)
_segsum_flow = _make_seg_sum(C=5120, P=5)
_segsum_host = _make_seg_sum(C=5120, P=1)


def _after(arr, dep):
    z = (dep.reshape(-1)[0] * 0).astype(arr.dtype)
    return arr + z


def kernel(x_host, x_flow, src_h2f, dst_h2f, src_f2h, dst_f2h,
           Wl_h2f_0, Wr_h2f_0, b_h2f_0, Wl_f2h_0, Wr_f2h_0, b_f2h_0,
           Wl_h2f_1, Wr_h2f_1, b_h2f_1, Wl_f2h_1, Wr_f2h_1, b_f2h_1,
           lin_W, lin_b):
    src_h2f = src_h2f.astype(jnp.int32)
    dst_h2f = dst_h2f.astype(jnp.int32)
    src_f2h = src_f2h.astype(jnp.int32)
    dst_f2h = dst_f2h.astype(jnp.int32)

    ones_host = jnp.ones((N_HOST, 128), jnp.float32)
    ones_flow = jnp.ones((N_FLOW, 128), jnp.float32)
    Scnt_f = _segsum_flow(ones_host, src_h2f, dst_h2f)
    cnt_f = Scnt_f[:, :16]
    Scnt_h = _segsum_host(ones_flow, _after(src_f2h, Scnt_f), dst_f2h)
    cnt_h = Scnt_h[:, :16]

    S_f0 = _segsum_flow(x_host, _after(src_h2f, Scnt_h), dst_h2f)

    xf1 = _combine1(S_f0, cnt_f, x_flow, Wl_h2f_0, Wr_h2f_0, b_h2f_0, True)

    S_h0 = _segsum_host(x_flow, _after(src_f2h, S_f0), dst_f2h)
    xh1 = _combine1(S_h0, cnt_h, x_host, Wl_f2h_0, Wr_f2h_0, b_f2h_0, True)

    xh1a = xh1[:, :128]
    xh1b = xh1[:, 128:]
    S_f1a = _segsum_flow(xh1a, _after(src_h2f, S_h0), dst_h2f)
    S_f1b = _segsum_flow(xh1b, _after(src_h2f, S_f1a), dst_h2f)
    xf2 = _combine2(S_f1a, S_f1b, cnt_f, xf1, Wl_h2f_1, Wr_h2f_1, b_h2f_1,
                    True)

    return _final(xf2, lin_W, lin_b)


# speedup vs baseline: 1.0259x; 1.0259x over previous
import functools

import jax
import jax.numpy as jnp
from jax import lax
from jax.experimental import pallas as pl
from jax.experimental.pallas import tpu as pltpu
from jax.experimental.pallas import tpu_sc as plsc

N_HOST = 10000
N_FLOW = 50000
E = 320000
D = 128
H = 256
OUT = 64

NC = 2
NS = 16

_MESH = plsc.VectorSubcoreMesh(
    core_axis_name="c", subcore_axis_name="s", num_cores=NC, num_subcores=NS)


def _make_counts(C, P):
    ept = E // NS
    nfull = ept // 128
    tail = ept - nfull * 128
    z_rpt = (C + 128) // NS
    d_rpt = C // NS

    def body(dst_hbm, out_hbm, dst_st, dfire, dfire_t, ones, zsrc, acc, sem):
        c = lax.axis_index("c")
        s = lax.axis_index("s")
        pltpu.sync_copy(dst_hbm.at[pl.ds(s * ept, ept)], dst_st)
        one16 = jnp.ones((16,), jnp.float32)
        zero16 = jnp.zeros((16,), jnp.float32)
        for r in range(128):
            ones[r, pl.ds(0, 16)] = one16
        for r in range(8):
            zsrc[r, pl.ds(0, 16)] = zero16
        it16 = lax.iota(jnp.int32, 16)

        for p in range(P):
            lo = (c * P + p) * C
            hi = lo + C
            _counts_pass(acc, dst_st, dfire, dfire_t, ones, zsrc, out_hbm,
                         s, lo, hi)

    def _counts_pass(acc, dst_st, dfire, dfire_t, ones, zsrc, out_hbm, s, lo, hi):
        it16 = lax.iota(jnp.int32, 16)
        one16 = jnp.ones((16,), jnp.float32)
        for r in range(128):
            ones[r, pl.ds(0, 16)] = one16

        def zb(i, _):
            pltpu.sync_copy(zsrc, acc.at[pl.ds(s * z_rpt + i * 8, 8)])
            return 0
        lax.fori_loop(0, z_rpt // 8, zb, 0)
        plsc.subcore_barrier()

        def build(df, ob, ngrp):
            for k in range(ngrp):
                d16 = dst_st[pl.ds(ob + k * 16, 16)]
                inr = (d16 >= lo) & (d16 < hi)
                df[pl.ds(k * 16, 16)] = jnp.where(inr, d16 - lo, C + it16)

        def blk(i, _):
            build(dfire, i * 128, 8)
            pltpu.sync_copy(ones, acc.at[dfire], add=True)
            return 0
        lax.fori_loop(0, nfull, blk, 0)
        if tail:
            build(dfire_t, nfull * 128, tail // 16)
            pltpu.sync_copy(ones.at[pl.ds(0, tail)], acc.at[dfire_t], add=True)
        plsc.subcore_barrier()

        dnfull = d_rpt // 128
        dtail = d_rpt - dnfull * 128
        for i in range(dnfull):
            r = s * d_rpt + i * 128
            pltpu.sync_copy(acc.at[pl.ds(r, 128)], ones)
            pltpu.sync_copy(ones, out_hbm.at[pl.ds(lo + r, 128)])
        if dtail:
            r = s * d_rpt + dnfull * 128
            pltpu.sync_copy(acc.at[pl.ds(r, dtail)], ones.at[pl.ds(0, dtail)])
            pltpu.sync_copy(ones.at[pl.ds(0, dtail)],
                            out_hbm.at[pl.ds(lo + r, dtail)])
        plsc.subcore_barrier()

    if tail % 16 or C % 128 or z_rpt % 8 or d_rpt % 8:
        raise ValueError("bad counts geometry")

    return pl.kernel(
        body,
        out_type=jax.ShapeDtypeStruct((NC * P * C, 16), jnp.float32),
        mesh=_MESH,
        scratch_types=[
            pltpu.VMEM((ept,), jnp.int32),
            pltpu.VMEM((128,), jnp.int32),
            pltpu.VMEM((max(tail, 16),), jnp.int32),
            pltpu.VMEM((128, 16), jnp.float32),
            pltpu.VMEM((8, 16), jnp.float32),
            pltpu.VMEM_SHARED((C + 128, 16), jnp.float32),
            pltpu.SemaphoreType.DMA,
        ],
    )


W = 128
FIRE = 128


def _make_seg_sum(C, P):
    ept = E // NS
    npair = ept // (2 * FIRE)
    tail = ept - npair * 2 * FIRE
    rows_out = NC * P * C
    if C % 128 or tail % 16 or tail >= FIRE:
        raise ValueError("bad segsum geometry")
    z_rpt = (C + 128) // NS
    d_rpt = C // NS

    def body(x_hbm, src_hbm, dst_hbm, out_hbm,
             src_st, dst_st, dfA, dfB, dfT, rowsA, rowsB, zsrc, acc,
             semA, semB, semSA, semSB):
        c = lax.axis_index("c")
        s = lax.axis_index("s")
        pltpu.sync_copy(src_hbm.at[pl.ds(s * ept, ept)], src_st)
        pltpu.sync_copy(dst_hbm.at[pl.ds(s * ept, ept)], dst_st)
        zero16 = jnp.zeros((16,), jnp.float32)
        for r in range(8):
            for k in range(W // 16):
                zsrc[r, pl.ds(k * 16, 16)] = zero16
        it16 = lax.iota(jnp.int32, 16)

        for p in range(P):
            lo = (c * P + p) * C
            hi = lo + C

            def zb(i, _):
                pltpu.sync_copy(zsrc, acc.at[pl.ds(s * z_rpt + i * 8, 8)])
                return 0
            lax.fori_loop(0, z_rpt // 8, zb, 0)
            plsc.subcore_barrier()

            def build(df, ob, ngrp):
                for k in range(ngrp):
                    d16 = dst_st[pl.ds(ob + k * 16, 16)]
                    inr = (d16 >= lo) & (d16 < hi)
                    df[pl.ds(k * 16, 16)] = jnp.where(inr, d16 - lo, C + it16)

            def pair(j, _):
                ob0 = j * (2 * FIRE)
                ob1 = ob0 + FIRE
                g0 = pltpu.async_copy(
                    x_hbm.at[src_st.at[pl.ds(ob0, FIRE)]], rowsA, semA)
                g1 = pltpu.async_copy(
                    x_hbm.at[src_st.at[pl.ds(ob1, FIRE)]], rowsB, semB)
                build(dfA, ob0, FIRE // 16)
                g0.wait()
                s0 = pltpu.async_copy(rowsA, acc.at[dfA], semSA, add=True)
                build(dfB, ob1, FIRE // 16)
                g1.wait()
                s1 = pltpu.async_copy(rowsB, acc.at[dfB], semSB, add=True)
                s0.wait()
                s1.wait()
                return 0
            lax.fori_loop(0, npair, pair, 0)
            if tail:
                ob = npair * 2 * FIRE
                gt = pltpu.async_copy(
                    x_hbm.at[src_st.at[pl.ds(ob, tail)]],
                    rowsA.at[pl.ds(0, tail)], semA)
                build(dfT, ob, tail // 16)
                gt.wait()
                pltpu.sync_copy(rowsA.at[pl.ds(0, tail)], acc.at[dfT], add=True)
            plsc.subcore_barrier()

            dnfull = d_rpt // FIRE
            dtail = d_rpt - dnfull * FIRE
            for i in range(dnfull):
                r = s * d_rpt + i * FIRE
                pltpu.sync_copy(acc.at[pl.ds(r, FIRE)], rowsA)
                pltpu.sync_copy(rowsA, out_hbm.at[pl.ds(lo + r, FIRE)])
            if dtail:
                r = s * d_rpt + dnfull * FIRE
                pltpu.sync_copy(acc.at[pl.ds(r, dtail)], rowsA.at[pl.ds(0, dtail)])
                pltpu.sync_copy(rowsA.at[pl.ds(0, dtail)],
                                out_hbm.at[pl.ds(lo + r, dtail)])
            plsc.subcore_barrier()

    return pl.kernel(
        body,
        out_type=jax.ShapeDtypeStruct((rows_out, W), jnp.float32),
        mesh=_MESH,
        scratch_types=[
            pltpu.VMEM((ept,), jnp.int32),
            pltpu.VMEM((ept,), jnp.int32),
            pltpu.VMEM((FIRE,), jnp.int32),
            pltpu.VMEM((FIRE,), jnp.int32),
            pltpu.VMEM((max(tail, 16),), jnp.int32),
            pltpu.VMEM((FIRE, W), jnp.float32),
            pltpu.VMEM((FIRE, W), jnp.float32),
            pltpu.VMEM((8, W), jnp.float32),
            pltpu.VMEM_SHARED((C + 128, W), jnp.float32),
            pltpu.SemaphoreType.DMA,
            pltpu.SemaphoreType.DMA,
            pltpu.SemaphoreType.DMA,
            pltpu.SemaphoreType.DMA,
        ],
    )


def _combine1_body(relu, s_ref, c_ref, x_ref, wl_ref, wr_ref, b_ref, o_ref):
    scale = 1.0 / jnp.maximum(c_ref[:, 0:1], 1.0)
    a = s_ref[...] * scale
    y = (jnp.dot(a, wl_ref[...], preferred_element_type=jnp.float32)
         + jnp.dot(x_ref[...], wr_ref[...], preferred_element_type=jnp.float32)
         + b_ref[...])
    o_ref[...] = jnp.where(y > 0, y, 0.01 * y) if relu else y


def _combine1(S_pad, cnt_pad, X, Wl, Wr, b, relu, BM=400):
    M, K = X.shape
    return pl.pallas_call(
        functools.partial(_combine1_body, relu),
        grid=(M // BM,),
        in_specs=[
            pl.BlockSpec((BM, K), lambda i: (i, 0)),
            pl.BlockSpec((BM, 16), lambda i: (i, 0)),
            pl.BlockSpec((BM, K), lambda i: (i, 0)),
            pl.BlockSpec((K, H), lambda i: (0, 0)),
            pl.BlockSpec((K, H), lambda i: (0, 0)),
            pl.BlockSpec((H,), lambda i: (0,)),
        ],
        out_specs=pl.BlockSpec((BM, H), lambda i: (i, 0)),
        out_shape=jax.ShapeDtypeStruct((M, H), jnp.float32),
    )(S_pad, cnt_pad, X, Wl, Wr, b)


def _combine2_body(relu, sa_ref, sb_ref, c_ref, x_ref, wl_ref, wr_ref, b_ref,
                   o_ref):
    scale = 1.0 / jnp.maximum(c_ref[:, 0:1], 1.0)
    a = jnp.concatenate([sa_ref[...], sb_ref[...]], axis=1) * scale
    y = (jnp.dot(a, wl_ref[...], preferred_element_type=jnp.float32)
         + jnp.dot(x_ref[...], wr_ref[...], preferred_element_type=jnp.float32)
         + b_ref[...])
    o_ref[...] = jnp.where(y > 0, y, 0.01 * y) if relu else y


def _combine2(Sa, Sb, cnt_pad, X, Wl, Wr, b, relu, BM=400):
    M, K = X.shape
    return pl.pallas_call(
        functools.partial(_combine2_body, relu),
        grid=(M // BM,),
        in_specs=[
            pl.BlockSpec((BM, W), lambda i: (i, 0)),
            pl.BlockSpec((BM, W), lambda i: (i, 0)),
            pl.BlockSpec((BM, 16), lambda i: (i, 0)),
            pl.BlockSpec((BM, K), lambda i: (i, 0)),
            pl.BlockSpec((K, H), lambda i: (0, 0)),
            pl.BlockSpec((K, H), lambda i: (0, 0)),
            pl.BlockSpec((H,), lambda i: (0,)),
        ],
        out_specs=pl.BlockSpec((BM, H), lambda i: (i, 0)),
        out_shape=jax.ShapeDtypeStruct((M, H), jnp.float32),
    )(Sa, Sb, cnt_pad, X, Wl, Wr, b)


def _final_body(x_ref, w_ref, b_ref, o_ref):
    o_ref[...] = jnp.dot(x_ref[...], w_ref[...],
                         preferred_element_type=jnp.float32) + b_ref[...]


def _final(x, w, b, BM=1000):
    M, K = x.shape
    return pl.pallas_call(
        _final_body,
        grid=(M // BM,),
        in_specs=[
            pl.BlockSpec((BM, K), lambda i: (i, 0)),
            pl.BlockSpec((K, OUT), lambda i: (0, 0)),
            pl.BlockSpec((OUT,), lambda i: (0,)),
        ],
        out_specs=pl.BlockSpec((BM, OUT), lambda i: (i, 0)),
        out_shape=jax.ShapeDtypeStruct((M, OUT), jnp.float32),
    )(x, w, b)


_counts_part = _make_counts(C=5120, P=1)
_counts_host = _make_counts(C=5120, P=1)
_segsum_flow = _make_seg_sum(C=5120, P=5)
_segsum_host = _make_seg_sum(C=5120, P=1)


def _after(arr, dep):
    z = (dep.reshape(-1)[0] * 0).astype(arr.dtype)
    return arr + z


def kernel(x_host, x_flow, src_h2f, dst_h2f, src_f2h, dst_f2h,
           Wl_h2f_0, Wr_h2f_0, b_h2f_0, Wl_f2h_0, Wr_f2h_0, b_f2h_0,
           Wl_h2f_1, Wr_h2f_1, b_h2f_1, Wl_f2h_1, Wr_f2h_1, b_f2h_1,
           lin_W, lin_b):
    src_h2f = src_h2f.astype(jnp.int32)
    dst_h2f = dst_h2f.astype(jnp.int32)
    src_f2h = src_f2h.astype(jnp.int32)
    dst_f2h = dst_f2h.astype(jnp.int32)

    ones_host = jnp.ones((N_HOST, 128), jnp.float32)
    ones_flow = jnp.ones((N_FLOW, 128), jnp.float32)
    Scnt_f = _segsum_flow(ones_host, src_h2f, dst_h2f)
    cnt_f = Scnt_f[:, :16]
    Scnt_h = _segsum_host(ones_flow, _after(src_f2h, Scnt_f), dst_f2h)
    cnt_h = Scnt_h[:, :16]

    S_f0 = _segsum_flow(x_host, _after(src_h2f, Scnt_h), dst_h2f)

    xf1 = _combine1(S_f0, cnt_f, x_flow, Wl_h2f_0, Wr_h2f_0, b_h2f_0, True)

    S_h0 = _segsum_host(x_flow, _after(src_f2h, S_f0), dst_f2h)
    xh1 = _combine1(S_h0, cnt_h, x_host, Wl_f2h_0, Wr_f2h_0, b_f2h_0, True)

    xh1a = xh1[:, :128]
    xh1b = xh1[:, 128:]
    S_f1a = _segsum_flow(xh1a, _after(src_h2f, S_h0), dst_h2f)
    S_f1b = _segsum_flow(xh1b, _after(src_h2f, S_f1a), dst_h2f)
    xf2 = _combine2(S_f1a, S_f1b, cnt_f, xf1, Wl_h2f_1, Wr_h2f_1, b_h2f_1,
                    True)

    return _final(xf2, lin_W, lin_b)


# speedup vs baseline: 1.2296x; 1.1985x over previous
import functools

import jax
import jax.numpy as jnp
from jax import lax
from jax.experimental import pallas as pl
from jax.experimental.pallas import tpu as pltpu
from jax.experimental.pallas import tpu_sc as plsc

N_HOST = 10000
N_FLOW = 50000
E = 320000
D = 128
H = 256
OUT = 64

NC = 2
NS = 16

_MESH = plsc.VectorSubcoreMesh(
    core_axis_name="c", subcore_axis_name="s", num_cores=NC, num_subcores=NS)


def _make_counts(C, P):
    ept = E // NS
    nfull = ept // 128
    tail = ept - nfull * 128
    z_rpt = (C + 128) // NS
    d_rpt = C // NS

    def body(dst_hbm, out_hbm, dst_st, dfire, dfire_t, ones, zsrc, acc, sem):
        c = lax.axis_index("c")
        s = lax.axis_index("s")
        pltpu.sync_copy(dst_hbm.at[pl.ds(s * ept, ept)], dst_st)
        one16 = jnp.ones((16,), jnp.float32)
        zero16 = jnp.zeros((16,), jnp.float32)
        for r in range(128):
            ones[r, pl.ds(0, 16)] = one16
        for r in range(8):
            zsrc[r, pl.ds(0, 16)] = zero16
        it16 = lax.iota(jnp.int32, 16)

        for p in range(P):
            lo = (c * P + p) * C
            hi = lo + C
            _counts_pass(acc, dst_st, dfire, dfire_t, ones, zsrc, out_hbm,
                         s, lo, hi)

    def _counts_pass(acc, dst_st, dfire, dfire_t, ones, zsrc, out_hbm, s, lo, hi):
        it16 = lax.iota(jnp.int32, 16)
        one16 = jnp.ones((16,), jnp.float32)
        for r in range(128):
            ones[r, pl.ds(0, 16)] = one16

        def zb(i, _):
            pltpu.sync_copy(zsrc, acc.at[pl.ds(s * z_rpt + i * 8, 8)])
            return 0
        lax.fori_loop(0, z_rpt // 8, zb, 0)
        plsc.subcore_barrier()

        def build(df, ob, ngrp):
            for k in range(ngrp):
                d16 = dst_st[pl.ds(ob + k * 16, 16)]
                inr = (d16 >= lo) & (d16 < hi)
                df[pl.ds(k * 16, 16)] = jnp.where(inr, d16 - lo, C + it16)

        def blk(i, _):
            build(dfire, i * 128, 8)
            pltpu.sync_copy(ones, acc.at[dfire], add=True)
            return 0
        lax.fori_loop(0, nfull, blk, 0)
        if tail:
            build(dfire_t, nfull * 128, tail // 16)
            pltpu.sync_copy(ones.at[pl.ds(0, tail)], acc.at[dfire_t], add=True)
        plsc.subcore_barrier()

        dnfull = d_rpt // 128
        dtail = d_rpt - dnfull * 128
        for i in range(dnfull):
            r = s * d_rpt + i * 128
            pltpu.sync_copy(acc.at[pl.ds(r, 128)], ones)
            pltpu.sync_copy(ones, out_hbm.at[pl.ds(lo + r, 128)])
        if dtail:
            r = s * d_rpt + dnfull * 128
            pltpu.sync_copy(acc.at[pl.ds(r, dtail)], ones.at[pl.ds(0, dtail)])
            pltpu.sync_copy(ones.at[pl.ds(0, dtail)],
                            out_hbm.at[pl.ds(lo + r, dtail)])
        plsc.subcore_barrier()

    if tail % 16 or C % 128 or z_rpt % 8 or d_rpt % 8:
        raise ValueError("bad counts geometry")

    return pl.kernel(
        body,
        out_type=jax.ShapeDtypeStruct((NC * P * C, 16), jnp.float32),
        mesh=_MESH,
        scratch_types=[
            pltpu.VMEM((ept,), jnp.int32),
            pltpu.VMEM((128,), jnp.int32),
            pltpu.VMEM((max(tail, 16),), jnp.int32),
            pltpu.VMEM((128, 16), jnp.float32),
            pltpu.VMEM((8, 16), jnp.float32),
            pltpu.VMEM_SHARED((C + 128, 16), jnp.float32),
            pltpu.SemaphoreType.DMA,
        ],
    )


W = 128
FIRE = 128


def _make_seg_sum(C, P):
    ept = E // NS
    npair = ept // (2 * FIRE)
    tail = ept - npair * 2 * FIRE
    rows_out = NC * P * C
    if C % 128 or tail % 16 or tail >= FIRE:
        raise ValueError("bad segsum geometry")
    z_rpt = (C + 128) // NS
    d_rpt = C // NS

    def body(x_hbm, src_hbm, dst_hbm, out_hbm,
             src_st, dst_st, dfA, dfB, dfT, rowsA, rowsB, zsrc, acc,
             semA, semB, semSA, semSB):
        c = lax.axis_index("c")
        s = lax.axis_index("s")
        pltpu.sync_copy(src_hbm.at[pl.ds(s * ept, ept)], src_st)
        pltpu.sync_copy(dst_hbm.at[pl.ds(s * ept, ept)], dst_st)
        zero16 = jnp.zeros((16,), jnp.float32)
        for r in range(8):
            for k in range(W // 16):
                zsrc[r, pl.ds(k * 16, 16)] = zero16
        it16 = lax.iota(jnp.int32, 16)

        for p in range(P):
            lo = (c * P + p) * C
            hi = lo + C

            def zb(i, _):
                pltpu.sync_copy(zsrc, acc.at[pl.ds(s * z_rpt + i * 8, 8)])
                return 0
            lax.fori_loop(0, z_rpt // 8, zb, 0)
            plsc.subcore_barrier()

            def build(df, ob, ngrp):
                for k in range(ngrp):
                    d16 = dst_st[pl.ds(ob + k * 16, 16)]
                    inr = (d16 >= lo) & (d16 < hi)
                    df[pl.ds(k * 16, 16)] = jnp.where(inr, d16 - lo, C + it16)

            def pair(j, _):
                ob0 = j * (2 * FIRE)
                ob1 = ob0 + FIRE
                g0 = pltpu.async_copy(
                    x_hbm.at[src_st.at[pl.ds(ob0, FIRE)]], rowsA, semA)
                g1 = pltpu.async_copy(
                    x_hbm.at[src_st.at[pl.ds(ob1, FIRE)]], rowsB, semB)
                build(dfA, ob0, FIRE // 16)
                g0.wait()
                s0 = pltpu.async_copy(rowsA, acc.at[dfA], semSA, add=True)
                build(dfB, ob1, FIRE // 16)
                g1.wait()
                s1 = pltpu.async_copy(rowsB, acc.at[dfB], semSB, add=True)
                s0.wait()
                s1.wait()
                return 0
            lax.fori_loop(0, npair, pair, 0)
            if tail:
                ob = npair * 2 * FIRE
                gt = pltpu.async_copy(
                    x_hbm.at[src_st.at[pl.ds(ob, tail)]],
                    rowsA.at[pl.ds(0, tail)], semA)
                build(dfT, ob, tail // 16)
                gt.wait()
                pltpu.sync_copy(rowsA.at[pl.ds(0, tail)], acc.at[dfT], add=True)
            plsc.subcore_barrier()

            dnfull = d_rpt // FIRE
            dtail = d_rpt - dnfull * FIRE
            for i in range(dnfull):
                r = s * d_rpt + i * FIRE
                pltpu.sync_copy(acc.at[pl.ds(r, FIRE)], rowsA)
                pltpu.sync_copy(rowsA, out_hbm.at[pl.ds(lo + r, FIRE)])
            if dtail:
                r = s * d_rpt + dnfull * FIRE
                pltpu.sync_copy(acc.at[pl.ds(r, dtail)], rowsA.at[pl.ds(0, dtail)])
                pltpu.sync_copy(rowsA.at[pl.ds(0, dtail)],
                                out_hbm.at[pl.ds(lo + r, dtail)])
            plsc.subcore_barrier()

    return pl.kernel(
        body,
        out_type=jax.ShapeDtypeStruct((rows_out, W), jnp.float32),
        mesh=_MESH,
        scratch_types=[
            pltpu.VMEM((ept,), jnp.int32),
            pltpu.VMEM((ept,), jnp.int32),
            pltpu.VMEM((FIRE,), jnp.int32),
            pltpu.VMEM((FIRE,), jnp.int32),
            pltpu.VMEM((max(tail, 16),), jnp.int32),
            pltpu.VMEM((FIRE, W), jnp.float32),
            pltpu.VMEM((FIRE, W), jnp.float32),
            pltpu.VMEM((8, W), jnp.float32),
            pltpu.VMEM_SHARED((C + 128, W), jnp.float32),
            pltpu.SemaphoreType.DMA,
            pltpu.SemaphoreType.DMA,
            pltpu.SemaphoreType.DMA,
            pltpu.SemaphoreType.DMA,
        ],
    )


def _combine1_body(relu, s_ref, c_ref, x_ref, wl_ref, wr_ref, b_ref, o_ref):
    scale = 1.0 / jnp.maximum(c_ref[:, 0:1], 1.0)
    a = s_ref[...] * scale
    y = (jnp.dot(a, wl_ref[...], preferred_element_type=jnp.float32)
         + jnp.dot(x_ref[...], wr_ref[...], preferred_element_type=jnp.float32)
         + b_ref[...])
    o_ref[...] = jnp.where(y > 0, y, 0.01 * y) if relu else y


def _combine1(S_pad, cnt_pad, X, Wl, Wr, b, relu, BM=400):
    M, K = X.shape
    return pl.pallas_call(
        functools.partial(_combine1_body, relu),
        grid=(M // BM,),
        in_specs=[
            pl.BlockSpec((BM, K), lambda i: (i, 0)),
            pl.BlockSpec((BM, 16), lambda i: (i, 0)),
            pl.BlockSpec((BM, K), lambda i: (i, 0)),
            pl.BlockSpec((K, H), lambda i: (0, 0)),
            pl.BlockSpec((K, H), lambda i: (0, 0)),
            pl.BlockSpec((H,), lambda i: (0,)),
        ],
        out_specs=pl.BlockSpec((BM, H), lambda i: (i, 0)),
        out_shape=jax.ShapeDtypeStruct((M, H), jnp.float32),
    )(S_pad, cnt_pad, X, Wl, Wr, b)


def _combine2_body(relu, sa_ref, sb_ref, c_ref, x_ref, wl_ref, wr_ref, b_ref,
                   o_ref):
    scale = 1.0 / jnp.maximum(c_ref[:, 0:1], 1.0)
    a = jnp.concatenate([sa_ref[...], sb_ref[...]], axis=1) * scale
    y = (jnp.dot(a, wl_ref[...], preferred_element_type=jnp.float32)
         + jnp.dot(x_ref[...], wr_ref[...], preferred_element_type=jnp.float32)
         + b_ref[...])
    o_ref[...] = jnp.where(y > 0, y, 0.01 * y) if relu else y


def _combine2(Sa, Sb, cnt_pad, X, Wl, Wr, b, relu, BM=400):
    M, K = X.shape
    return pl.pallas_call(
        functools.partial(_combine2_body, relu),
        grid=(M // BM,),
        in_specs=[
            pl.BlockSpec((BM, W), lambda i: (i, 0)),
            pl.BlockSpec((BM, W), lambda i: (i, 0)),
            pl.BlockSpec((BM, 16), lambda i: (i, 0)),
            pl.BlockSpec((BM, K), lambda i: (i, 0)),
            pl.BlockSpec((K, H), lambda i: (0, 0)),
            pl.BlockSpec((K, H), lambda i: (0, 0)),
            pl.BlockSpec((H,), lambda i: (0,)),
        ],
        out_specs=pl.BlockSpec((BM, H), lambda i: (i, 0)),
        out_shape=jax.ShapeDtypeStruct((M, H), jnp.float32),
    )(Sa, Sb, cnt_pad, X, Wl, Wr, b)


def _final_body(x_ref, w_ref, b_ref, o_ref):
    o_ref[...] = jnp.dot(x_ref[...], w_ref[...],
                         preferred_element_type=jnp.float32) + b_ref[...]


def _final(x, w, b, BM=1000):
    M, K = x.shape
    return pl.pallas_call(
        _final_body,
        grid=(M // BM,),
        in_specs=[
            pl.BlockSpec((BM, K), lambda i: (i, 0)),
            pl.BlockSpec((K, OUT), lambda i: (0, 0)),
            pl.BlockSpec((OUT,), lambda i: (0,)),
        ],
        out_specs=pl.BlockSpec((BM, OUT), lambda i: (i, 0)),
        out_shape=jax.ShapeDtypeStruct((M, OUT), jnp.float32),
    )(x, w, b)


_counts_part = _make_counts(C=5120, P=1)
_counts_host = _make_counts(C=5120, P=1)
_segsum_flow = _make_seg_sum(C=6912, P=4)
_segsum_host = _make_seg_sum(C=5120, P=1)


def _after(arr, dep):
    z = (dep.reshape(-1)[0] * 0).astype(arr.dtype)
    return arr + z


def kernel(x_host, x_flow, src_h2f, dst_h2f, src_f2h, dst_f2h,
           Wl_h2f_0, Wr_h2f_0, b_h2f_0, Wl_f2h_0, Wr_f2h_0, b_f2h_0,
           Wl_h2f_1, Wr_h2f_1, b_h2f_1, Wl_f2h_1, Wr_f2h_1, b_f2h_1,
           lin_W, lin_b):
    src_h2f = src_h2f.astype(jnp.int32)
    dst_h2f = dst_h2f.astype(jnp.int32)
    src_f2h = src_f2h.astype(jnp.int32)
    dst_f2h = dst_f2h.astype(jnp.int32)

    ones_host = jnp.ones((N_HOST, 128), jnp.float32)
    ones_flow = jnp.ones((N_FLOW, 128), jnp.float32)
    Scnt_f = _segsum_flow(ones_host, src_h2f, dst_h2f)
    cnt_f = Scnt_f[:, :16]
    Scnt_h = _segsum_host(ones_flow, _after(src_f2h, Scnt_f), dst_f2h)
    cnt_h = Scnt_h[:, :16]

    S_f0 = _segsum_flow(x_host, _after(src_h2f, Scnt_h), dst_h2f)

    xf1 = _combine1(S_f0, cnt_f, x_flow, Wl_h2f_0, Wr_h2f_0, b_h2f_0, True)

    S_h0 = _segsum_host(x_flow, _after(src_f2h, S_f0), dst_f2h)
    xh1 = _combine1(S_h0, cnt_h, x_host, Wl_f2h_0, Wr_f2h_0, b_f2h_0, True)

    xh1a = xh1[:, :128]
    xh1b = xh1[:, 128:]
    S_f1a = _segsum_flow(xh1a, _after(src_h2f, S_h0), dst_h2f)
    S_f1b = _segsum_flow(xh1b, _after(src_h2f, S_f1a), dst_h2f)
    xf2 = _combine2(S_f1a, S_f1b, cnt_f, xf1, Wl_h2f_1, Wr_h2f_1, b_h2f_1,
                    True)

    return _final(xf2, lin_W, lin_b)
